# baseline (device time: 22943 ns/iter reference)
import jax
import jax.numpy as jnp
from jax import lax
from jax.experimental import pallas as pl
from jax.experimental.pallas import tpu as pltpu

N_DEV = 16


def kernel(x, w_mat):
    m_per, k = x.shape
    n = w_mat.shape[1]
    n_per = n // N_DEV
    m = m_per * N_DEV

    def body(x_ref, w_ref, out_ref, send_buf, gather_ref,
             send_sems, recv_sems):
        p = lax.axis_index("i")

        barrier_sem = pltpu.get_barrier_semaphore()
        for d in range(1, N_DEV):
            peer = lax.rem(p + d, N_DEV)
            pl.semaphore_signal(barrier_sem, inc=1, device_id=(peer,),
                                device_id_type=pl.DeviceIdType.MESH)

        order = sorted(range(1, N_DEV), key=lambda d: -min(d, N_DEV - d))
        for c, d_off in enumerate(order + [0]):
            dst = lax.rem(p + d_off, N_DEV)
            wcol = w_ref[:, pl.ds(dst * n_per, n_per)]
            z = jnp.dot(x_ref[...], wcol,
                        preferred_element_type=jnp.float32)
            z = jnp.maximum(z, 0.0).astype(jnp.bfloat16)
            if d_off != 0:
                send_buf[c] = z
                if c == 0:
                    pl.semaphore_wait(barrier_sem, N_DEV - 1)
                rdma = pltpu.make_async_remote_copy(
                    src_ref=send_buf.at[c],
                    dst_ref=gather_ref.at[pl.ds(p * m_per, m_per), :],
                    send_sem=send_sems.at[c],
                    recv_sem=recv_sems.at[c],
                    device_id=(dst,),
                    device_id_type=pl.DeviceIdType.MESH,
                )
                rdma.start()
            else:
                out_ref[pl.ds(p * m_per, m_per), :] = z.astype(jnp.float32)

        for c in range(N_DEV - 1):
            d_off = order[c]
            src = lax.rem(p - d_off + N_DEV, N_DEV)
            recv = pltpu.make_async_remote_copy(
                src_ref=send_buf.at[c],
                dst_ref=gather_ref.at[pl.ds(src * m_per, m_per), :],
                send_sem=send_sems.at[c],
                recv_sem=recv_sems.at[c],
                device_id=(src,),
                device_id_type=pl.DeviceIdType.MESH,
            )
            recv.wait_recv()
            out_ref[pl.ds(src * m_per, m_per), :] = (
                gather_ref[pl.ds(src * m_per, m_per), :].astype(jnp.float32)
            )
        for c in range(N_DEV - 1):
            dst = lax.rem(p + order[c], N_DEV)
            send = pltpu.make_async_remote_copy(
                src_ref=send_buf.at[c],
                dst_ref=gather_ref.at[pl.ds(p * m_per, m_per), :],
                send_sem=send_sems.at[c],
                recv_sem=recv_sems.at[c],
                device_id=(dst,),
                device_id_type=pl.DeviceIdType.MESH,
            )
            send.wait_send()

    return pl.pallas_call(
        body,
        out_shape=jax.ShapeDtypeStruct((m, n_per), jnp.float32),
        in_specs=[
            pl.BlockSpec(memory_space=pltpu.VMEM),
            pl.BlockSpec(memory_space=pltpu.VMEM),
        ],
        out_specs=pl.BlockSpec(memory_space=pltpu.VMEM),
        scratch_shapes=[
            pltpu.VMEM((N_DEV - 1, m_per, n_per), jnp.bfloat16),
            pltpu.VMEM((m, n_per), jnp.bfloat16),
            pltpu.SemaphoreType.DMA((N_DEV - 1,)),
            pltpu.SemaphoreType.DMA((N_DEV - 1,)),
        ],
        compiler_params=pltpu.CompilerParams(collective_id=0),
    )(x.astype(jnp.bfloat16), w_mat.astype(jnp.bfloat16))


# device time: 21921 ns/iter; 1.0466x vs baseline; 1.0466x over previous
import jax
import jax.numpy as jnp
from jax import lax
from jax.experimental import pallas as pl
from jax.experimental.pallas import tpu as pltpu

N_DEV = 16


def kernel(x, w_mat):
    m_per, k = x.shape
    n = w_mat.shape[1]
    n_per = n // N_DEV
    m = m_per * N_DEV

    def body(x_ref, w_ref, out_ref, xb_ref, send_buf, gather_ref,
             send_sems, recv_sems):
        p = lax.axis_index("i")

        barrier_sem = pltpu.get_barrier_semaphore()
        for d in range(1, N_DEV):
            peer = lax.rem(p + d, N_DEV)
            pl.semaphore_signal(barrier_sem, inc=1, device_id=(peer,),
                                device_id_type=pl.DeviceIdType.MESH)

        xb_ref[...] = x_ref[...].astype(jnp.bfloat16)

        order = sorted(range(1, N_DEV), key=lambda d: -min(d, N_DEV - d))
        for c, d_off in enumerate(order + [0]):
            dst = lax.rem(p + d_off, N_DEV)
            wcol = w_ref[:, pl.ds(dst * n_per, n_per)]
            z = jnp.dot(xb_ref[...], wcol.astype(jnp.bfloat16),
                        preferred_element_type=jnp.float32)
            z = jnp.maximum(z, 0.0).astype(jnp.bfloat16)
            if d_off != 0:
                send_buf[c] = z
                if c == 0:
                    pl.semaphore_wait(barrier_sem, N_DEV - 1)
                rdma = pltpu.make_async_remote_copy(
                    src_ref=send_buf.at[c],
                    dst_ref=gather_ref.at[pl.ds(p * m_per, m_per), :],
                    send_sem=send_sems.at[c],
                    recv_sem=recv_sems.at[c],
                    device_id=(dst,),
                    device_id_type=pl.DeviceIdType.MESH,
                )
                rdma.start()
            else:
                out_ref[pl.ds(p * m_per, m_per), :] = z.astype(jnp.float32)

        for c in range(N_DEV - 1):
            d_off = order[c]
            src = lax.rem(p - d_off + N_DEV, N_DEV)
            recv = pltpu.make_async_remote_copy(
                src_ref=send_buf.at[c],
                dst_ref=gather_ref.at[pl.ds(src * m_per, m_per), :],
                send_sem=send_sems.at[c],
                recv_sem=recv_sems.at[c],
                device_id=(src,),
                device_id_type=pl.DeviceIdType.MESH,
            )
            recv.wait_recv()
            out_ref[pl.ds(src * m_per, m_per), :] = (
                gather_ref[pl.ds(src * m_per, m_per), :].astype(jnp.float32)
            )
        for c in range(N_DEV - 1):
            dst = lax.rem(p + order[c], N_DEV)
            send = pltpu.make_async_remote_copy(
                src_ref=send_buf.at[c],
                dst_ref=gather_ref.at[pl.ds(p * m_per, m_per), :],
                send_sem=send_sems.at[c],
                recv_sem=recv_sems.at[c],
                device_id=(dst,),
                device_id_type=pl.DeviceIdType.MESH,
            )
            send.wait_send()

    return pl.pallas_call(
        body,
        out_shape=jax.ShapeDtypeStruct((m, n_per), jnp.float32),
        in_specs=[
            pl.BlockSpec(memory_space=pltpu.VMEM),
            pl.BlockSpec(memory_space=pltpu.VMEM),
        ],
        out_specs=pl.BlockSpec(memory_space=pltpu.VMEM),
        scratch_shapes=[
            pltpu.VMEM((m_per, k), jnp.bfloat16),
            pltpu.VMEM((N_DEV - 1, m_per, n_per), jnp.bfloat16),
            pltpu.VMEM((m, n_per), jnp.bfloat16),
            pltpu.SemaphoreType.DMA((N_DEV - 1,)),
            pltpu.SemaphoreType.DMA((N_DEV - 1,)),
        ],
        compiler_params=pltpu.CompilerParams(collective_id=0),
    )(x, w_mat)
